# Initial kernel scaffold; baseline (speedup 1.0000x reference)
import jax
import jax.numpy as jnp
from jax.experimental import pallas as pl


def kernel(q_sub, q_rel, hidden, edges, nodes, old_nodes_new_idx, batchsize, rela_embed, Ws, Wr, Wqr, bqr, wa_w, wa_b, Wh):
    raise NotImplementedError("write your pallas kernel here")


# speedup vs baseline: 8.6600x; 8.6600x over previous
import functools

import jax
import jax.numpy as jnp
from jax import lax
from jax.experimental import pallas as pl
from jax.experimental.pallas import tpu as pltpu
from jax.experimental.pallas import tpu_sc as plsc

_NC = 2
_NS = 16
_NW = _NC * _NS

_SUB = 125
_CHUNK = _SUB


def _t1_body(h_ref, ws_ref, o_ref):
    h = h_ref[...]
    proj = lax.dot_general(h, ws_ref[...], (((1,), (1,)), ((), ())),
                           preferred_element_type=jnp.float32)
    o_ref[...] = jnp.concatenate([h, proj], axis=1)


def _t23_body(r_ref, wr_ref, bqr_ref, wqr_ref, t2_ref, t3_ref):
    r = r_ref[...]
    p2 = lax.dot_general(r, wr_ref[...], (((1,), (1,)), ((), ())),
                         preferred_element_type=jnp.float32) + bqr_ref[...]
    t2_ref[...] = jnp.concatenate([r, p2], axis=1)
    t3_ref[...] = lax.dot_general(r, wqr_ref[...], (((1,), (1,)), ((), ())),
                                  preferred_element_type=jnp.float32)


def _out_body(p_ref, wh_ref, o_ref):
    p = p_ref[...]
    m = p[0] + p[1]
    o_ref[...] = lax.dot_general(m, wh_ref[...], (((1,), (1,)), ((), ())),
                                 preferred_element_type=jnp.float32)


def _tq_kernel(t3_hbm, qrel_hbm, tq_hbm, idx_v, rows_v, sem):
    c = lax.axis_index("c")
    s = lax.axis_index("s")
    wid = s * _NC + c
    pltpu.sync_copy(qrel_hbm.at[wid], idx_v)
    cps = [pltpu.async_copy(t3_hbm.at[idx_v.at[j]],
                            rows_v.at[pl.ds(j * _SUB, _SUB)], sem)
           for j in range(16)]
    for cp in cps:
        cp.wait()
    pltpu.sync_copy(rows_v, tq_hbm.at[wid])


def _edge_kernel(n_node, n_chunks,
                 t1, t2, tq, sub_h, rel_h, rix_h, obj_h, w_h, zeros_h,
                 out_h,
                 sub_i, rel_i, rix_i, obj_i, sbuf, rbuf, qbuf, msg, wv, acc,
                 sem_i, sem_g):
    c = lax.axis_index("c")
    s = lax.axis_index("s")
    wid = s * _NC + c
    rows_per_tile = n_node // _NS

    pltpu.sync_copy(w_h, wv)
    pltpu.sync_copy(zeros_h.at[pl.ds(s * rows_per_tile, rows_per_tile)],
                    acc.at[pl.ds(s * rows_per_tile, rows_per_tile)])
    plsc.subcore_barrier()

    w0 = wv[pl.ds(0, 16)]
    w1 = wv[pl.ds(16, 16)]
    wb = wv[pl.ds(32, 16)]

    def chunk_body(k, carry):
        ci = wid * n_chunks + k
        cps = [pltpu.async_copy(sub_h.at[ci], sub_i, sem_i),
               pltpu.async_copy(rel_h.at[ci], rel_i, sem_i),
               pltpu.async_copy(rix_h.at[ci], rix_i, sem_i),
               pltpu.async_copy(obj_h.at[ci], obj_i, sem_i)]
        for cp in cps:
            cp.wait()
        gs = [pltpu.async_copy(t1.at[sub_i], sbuf, sem_g),
              pltpu.async_copy(t2.at[rel_i], rbuf, sem_g),
              pltpu.async_copy(tq.at[rix_i], qbuf, sem_g)]
        for g in gs:
            g.wait()

        @plsc.parallel_loop(0, _CHUNK, 1, unroll=5)
        def _(e):
            x0 = sbuf[e, pl.ds(32, 16)] + rbuf[e, pl.ds(32, 16)] \
                + qbuf[e, pl.ds(0, 16)]
            x1 = sbuf[e, pl.ds(48, 16)] + rbuf[e, pl.ds(48, 16)] \
                + qbuf[e, pl.ds(16, 16)]
            x0 = jnp.maximum(x0, 0.0)
            x1 = jnp.maximum(x1, 0.0)
            y = x0 * w0 + x1 * w1
            z = jnp.sum(y)
            zv = jnp.broadcast_to(z, (16,)) + wb
            av = 1.0 / (1.0 + jnp.exp(-zv))
            m0 = (sbuf[e, pl.ds(0, 16)] + rbuf[e, pl.ds(0, 16)]) * av
            m1 = (sbuf[e, pl.ds(16, 16)] + rbuf[e, pl.ds(16, 16)]) * av
            msg[e, pl.ds(0, 16)] = m0
            msg[e, pl.ds(16, 16)] = m1

        pltpu.sync_copy(msg, acc.at[obj_i], add=True)
        return carry

    lax.fori_loop(0, n_chunks, chunk_body, 0)
    plsc.subcore_barrier()
    pltpu.sync_copy(acc.at[pl.ds(s * rows_per_tile, rows_per_tile)],
                    out_h.at[c, pl.ds(s * rows_per_tile, rows_per_tile)])


def kernel(q_sub, q_rel, hidden, edges, nodes, old_nodes_new_idx, batchsize,
           rela_embed, Ws, Wr, Wqr, bqr, wa_w, wa_b, Wh):
    n_node, in_dim = hidden.shape
    n_edge = edges.shape[0]
    n_batch = q_rel.shape[0]
    n_rel = rela_embed.shape[0]

    assert in_dim == 32 and n_node % _NS == 0
    assert n_edge % (_NW * _CHUNK) == 0
    n_chunks = n_edge // (_NW * _CHUNK)

    blk = 2000
    rp = ((n_rel + blk - 1) // blk) * blk
    rela_p = jnp.pad(rela_embed, ((0, rp - n_rel), (0, 0)))

    t1 = pl.pallas_call(
        _t1_body,
        grid=(n_node // blk,),
        in_specs=[pl.BlockSpec((blk, in_dim), lambda i: (i, 0)),
                  pl.BlockSpec((in_dim, in_dim), lambda i: (0, 0))],
        out_specs=pl.BlockSpec((blk, 2 * in_dim), lambda i: (i, 0)),
        out_shape=jax.ShapeDtypeStruct((n_node, 2 * in_dim), jnp.float32),
    )(hidden, Ws)

    t2, t3 = pl.pallas_call(
        _t23_body,
        grid=(rp // blk,),
        in_specs=[pl.BlockSpec((blk, in_dim), lambda i: (i, 0)),
                  pl.BlockSpec((in_dim, in_dim), lambda i: (0, 0)),
                  pl.BlockSpec((1, in_dim), lambda i: (0, 0)),
                  pl.BlockSpec((in_dim, in_dim), lambda i: (0, 0))],
        out_specs=[pl.BlockSpec((blk, 2 * in_dim), lambda i: (i, 0)),
                   pl.BlockSpec((blk, in_dim), lambda i: (i, 0))],
        out_shape=[jax.ShapeDtypeStruct((rp, 2 * in_dim), jnp.float32),
                   jax.ShapeDtypeStruct((rp, in_dim), jnp.float32)],
    )(rela_p, Wr, bqr.reshape(1, in_dim), Wqr)

    bp = _NW * 16 * _SUB
    qrel_p = jnp.pad(q_rel, (0, bp - n_batch)).reshape(_NW, 16, _SUB)
    mesh = plsc.VectorSubcoreMesh(core_axis_name="c", subcore_axis_name="s")
    sc_params = pltpu.CompilerParams(use_tc_tiling_on_sc=False,
                                     needs_layout_passes=False)
    tq = pl.kernel(
        _tq_kernel,
        out_type=jax.ShapeDtypeStruct((_NW, 16 * _SUB, in_dim), jnp.float32),
        mesh=mesh,
        compiler_params=sc_params,
        scratch_types=[pltpu.VMEM((16, _SUB), jnp.int32),
                       pltpu.VMEM((16 * _SUB, in_dim), jnp.float32),
                       pltpu.SemaphoreType.DMA],
    )(t3, qrel_p)
    tq = tq.reshape(bp, in_dim)

    n_cidx = n_edge // _CHUNK
    sub_h = edges[:, 4].reshape(n_cidx, _SUB)
    rel_h = edges[:, 2].reshape(n_cidx, _SUB)
    rix_h = edges[:, 0].reshape(n_cidx, _SUB)
    obj_h = edges[:, 5].reshape(n_cidx, _SUB)
    wvec = jnp.concatenate([wa_w.reshape(-1),
                            jnp.broadcast_to(wa_b.reshape(-1), (16,))])
    zeros = jnp.zeros((n_node, in_dim), jnp.float32)

    partials = pl.kernel(
        functools.partial(_edge_kernel, n_node, n_chunks),
        out_type=jax.ShapeDtypeStruct((_NC, n_node, in_dim), jnp.float32),
        mesh=mesh,
        compiler_params=sc_params,
        scratch_types=[pltpu.VMEM((_SUB,), jnp.int32),
                       pltpu.VMEM((_SUB,), jnp.int32),
                       pltpu.VMEM((_SUB,), jnp.int32),
                       pltpu.VMEM((_SUB,), jnp.int32),
                       pltpu.VMEM((_CHUNK, 2 * in_dim), jnp.float32),
                       pltpu.VMEM((_CHUNK, 2 * in_dim), jnp.float32),
                       pltpu.VMEM((_CHUNK, in_dim), jnp.float32),
                       pltpu.VMEM((_CHUNK, in_dim), jnp.float32),
                       pltpu.VMEM((48,), jnp.float32),
                       pltpu.VMEM_SHARED((n_node, in_dim), jnp.float32),
                       pltpu.SemaphoreType.DMA,
                       pltpu.SemaphoreType.DMA],
    )(t1, t2, tq, sub_h, rel_h, rix_h, obj_h, wvec, zeros)

    out = pl.pallas_call(
        _out_body,
        grid=(n_node // blk,),
        in_specs=[pl.BlockSpec((_NC, blk, in_dim), lambda i: (0, i, 0)),
                  pl.BlockSpec((in_dim, in_dim), lambda i: (0, 0))],
        out_specs=pl.BlockSpec((blk, in_dim), lambda i: (i, 0)),
        out_shape=jax.ShapeDtypeStruct((n_node, in_dim), jnp.float32),
    )(partials, Wh)
    return out
